# baseline (speedup 1.0000x reference)
import functools

import jax
import jax.numpy as jnp
from jax import lax
from jax.experimental import pallas as pl
from jax.experimental.pallas import tpu as pltpu
from jax.experimental.pallas import tpu_sc as plsc

BTM = 256
CH = 16


def _routing_body(x_ref, gw_ref, e0_ref, e1_ref, r0_ref, r1_ref,
                  w0_ref, w1_ref, offs_ref, bmap_ref, cnt_ref):
    s = pl.program_id(0)
    S = x_ref.shape[0]
    E = gw_ref.shape[0]
    NBLKP = bmap_ref.shape[1]

    @pl.when(s == 0)
    def _():
        cnt_ref[...] = jnp.zeros_like(cnt_ref)

    x = x_ref[...]
    logits = lax.dot_general(x, gw_ref[...], (((1,), (1,)), ((), ())),
                             preferred_element_type=jnp.float32)
    iota = lax.broadcasted_iota(jnp.int32, (S, E), 1)
    m = jnp.max(logits, axis=-1, keepdims=True)
    p = jnp.exp(logits - m)
    m1 = jnp.max(p, axis=-1, keepdims=True)
    i1 = jnp.min(jnp.where(p == m1, iota, E), axis=-1, keepdims=True)
    mask1 = iota == i1
    p2 = jnp.where(mask1, -jnp.inf, p)
    m2 = jnp.max(p2, axis=-1, keepdims=True)
    i2 = jnp.min(jnp.where(p2 == m2, iota, E), axis=-1, keepdims=True)
    mask2 = iota == i2
    denom = m1 + m2
    w0_ref[...] = m1 / denom
    w1_ref[...] = m2 / denom
    e0_ref[...] = i1
    e1_ref[...] = i2

    oh = mask1.astype(jnp.float32) + mask2.astype(jnp.float32)
    tril = (lax.broadcasted_iota(jnp.int32, (S, S), 0) >
            lax.broadcasted_iota(jnp.int32, (S, S), 1)).astype(jnp.float32)
    csum = lax.dot_general(tril, oh, (((1,), (0,)), ((), ())),
                           preferred_element_type=jnp.float32)
    csum = csum + cnt_ref[...].astype(jnp.float32)
    r0_ref[...] = jnp.sum(jnp.where(mask1, csum, 0.0), axis=-1,
                          keepdims=True).astype(jnp.int32)
    r1_ref[...] = jnp.sum(jnp.where(mask2, csum, 0.0), axis=-1,
                          keepdims=True).astype(jnp.int32)

    cnt = cnt_ref[...] + jnp.sum(oh, axis=0, keepdims=True).astype(jnp.int32)
    cnt_ref[...] = cnt

    padded = ((cnt + (BTM - 1)) // BTM) * BTM
    upper = (lax.broadcasted_iota(jnp.int32, (E, E), 0) <
             lax.broadcasted_iota(jnp.int32, (E, E), 1)).astype(jnp.float32)
    offs = lax.dot_general(padded.astype(jnp.float32), upper,
                           (((1,), (0,)), ((), ())),
                           preferred_element_type=jnp.float32)
    offs_ref[...] = jnp.concatenate(
        [offs.astype(jnp.int32), jnp.zeros((1, 16 - E), jnp.int32)], axis=1)
    incl = offs + padded.astype(jnp.float32)
    total = jnp.sum(padded)

    lane = lax.broadcasted_iota(jnp.int32, (1, NBLKP), 1)
    pos = jnp.minimum(lane * BTM, total - BTM).astype(jnp.float32)
    bexp = jnp.zeros((1, NBLKP), jnp.int32)
    for e in range(E):
        incl_e = jnp.sum(jnp.where(iota[:1] == e, incl, 0.0), axis=-1,
                         keepdims=True)
        bexp = bexp + (pos >= incl_e).astype(jnp.int32)
    nblk_used = (total // BTM).astype(jnp.int32)
    bmap_ref[...] = jnp.where(lane == NBLKP - 1, nblk_used, bexp)


def _pos_body(e0_ref, e1_ref, r0_ref, r1_ref, offs_ref, p0_ref, p1_ref):
    To = e0_ref.shape[0]
    L = offs_ref.shape[1]
    lane = lax.broadcasted_iota(jnp.int32, (To, L), 1)
    off_b = offs_ref[...]
    p0_ref[...] = jnp.sum(jnp.where(lane == e0_ref[...], off_b, 0),
                          axis=-1, keepdims=True) + r0_ref[...]
    p1_ref[...] = jnp.sum(jnp.where(lane == e1_ref[...], off_b, 0),
                          axis=-1, keepdims=True) + r1_ref[...]


def _moe_body(bmap_ref, xs_ref, w1_ref, w3_ref, w2_ref, o_ref):
    b = pl.program_id(0)
    nused = bmap_ref[bmap_ref.shape[0] - 1]

    @pl.when(b < nused)
    def _():
        x = xs_ref[...]
        g = lax.dot_general(x, w1_ref[0], (((1,), (1,)), ((), ())),
                            preferred_element_type=jnp.float32)
        u = lax.dot_general(x, w3_ref[0], (((1,), (1,)), ((), ())),
                            preferred_element_type=jnp.float32)
        h = g * lax.logistic(g) * u
        o_ref[...] = lax.dot_general(h, w2_ref[0], (((1,), (1,)), ((), ())),
                                     preferred_element_type=jnp.float32)


def _combine_body(y0_ref, y1_ref, w0_ref, w1_ref, o_ref):
    o_ref[...] = w0_ref[...] * y0_ref[...] + w1_ref[...] * y1_ref[...]


def _sc_worker_id():
    return lax.axis_index("s") * 2 + lax.axis_index("c")


def _sc_dispatch_body(x_hbm, p0_hbm, p1_hbm, xs_hbm,
                      rows_v, i0_v, i1_v, sem):
    T = x_hbm.shape[0]
    wid = _sc_worker_id()
    per_w = T // 32
    for ci in range(per_w // CH):
        t0 = wid * per_w + ci * CH
        pltpu.sync_copy(x_hbm.at[pl.ds(t0, CH)], rows_v)
        pltpu.sync_copy(p0_hbm.at[pl.ds(t0, CH)], i0_v)
        pltpu.sync_copy(p1_hbm.at[pl.ds(t0, CH)], i1_v)
        pltpu.async_copy(rows_v, xs_hbm.at[i0_v[...]], sem).wait()
        pltpu.async_copy(rows_v, xs_hbm.at[i1_v[...]], sem).wait()


def _sc_combine_body(ys_hbm, p0_hbm, p1_hbm, y0_hbm, y1_hbm,
                     rows_v, i0_v, i1_v, sem):
    T = y0_hbm.shape[0]
    wid = _sc_worker_id()
    per_w = T // 32
    for ci in range(per_w // CH):
        t0 = wid * per_w + ci * CH
        pltpu.sync_copy(p0_hbm.at[pl.ds(t0, CH)], i0_v)
        pltpu.sync_copy(p1_hbm.at[pl.ds(t0, CH)], i1_v)
        pltpu.async_copy(ys_hbm.at[i0_v[...]], rows_v, sem).wait()
        pltpu.sync_copy(rows_v, y0_hbm.at[pl.ds(t0, CH)])
        pltpu.async_copy(ys_hbm.at[i1_v[...]], rows_v, sem).wait()
        pltpu.sync_copy(rows_v, y1_hbm.at[pl.ds(t0, CH)])


def kernel(hidden_states, gate_w, w1, w2, w3, num_global_tokens,
           max_num_tokens_per_gpu):
    T, D = hidden_states.shape
    E, FF, _ = w1.shape
    K = 2
    x = hidden_states.astype(jnp.float32)

    NBLK = (T * K) // BTM + E
    NBLKP = NBLK + 1
    NROWS = NBLK * BTM
    S = T // 2

    e0, e1, r0, r1, w0, w1c, offs, bmap = pl.pallas_call(
        _routing_body,
        grid=(T // S,),
        in_specs=[
            pl.BlockSpec((S, D), lambda s: (s, 0)),
            pl.BlockSpec((E, D), lambda s: (0, 0)),
        ],
        out_specs=[
            pl.BlockSpec((S, 1), lambda s: (s, 0)),
            pl.BlockSpec((S, 1), lambda s: (s, 0)),
            pl.BlockSpec((S, 1), lambda s: (s, 0)),
            pl.BlockSpec((S, 1), lambda s: (s, 0)),
            pl.BlockSpec((S, 1), lambda s: (s, 0)),
            pl.BlockSpec((S, 1), lambda s: (s, 0)),
            pl.BlockSpec((1, 16), lambda s: (0, 0)),
            pl.BlockSpec((1, NBLKP), lambda s: (0, 0)),
        ],
        out_shape=[
            jax.ShapeDtypeStruct((T, 1), jnp.int32),
            jax.ShapeDtypeStruct((T, 1), jnp.int32),
            jax.ShapeDtypeStruct((T, 1), jnp.int32),
            jax.ShapeDtypeStruct((T, 1), jnp.int32),
            jax.ShapeDtypeStruct((T, 1), jnp.float32),
            jax.ShapeDtypeStruct((T, 1), jnp.float32),
            jax.ShapeDtypeStruct((1, 16), jnp.int32),
            jax.ShapeDtypeStruct((1, NBLKP), jnp.int32),
        ],
        scratch_shapes=[pltpu.VMEM((1, E), jnp.int32)],
        compiler_params=pltpu.CompilerParams(
            dimension_semantics=("arbitrary",)),
    )(x, gate_w)

    p0, p1 = pl.pallas_call(
        _pos_body,
        grid=(1,),
        in_specs=[
            pl.BlockSpec((T, 1), lambda i: (0, 0)),
            pl.BlockSpec((T, 1), lambda i: (0, 0)),
            pl.BlockSpec((T, 1), lambda i: (0, 0)),
            pl.BlockSpec((T, 1), lambda i: (0, 0)),
            pl.BlockSpec((1, 16), lambda i: (0, 0)),
        ],
        out_specs=[
            pl.BlockSpec((T, 1), lambda i: (0, 0)),
            pl.BlockSpec((T, 1), lambda i: (0, 0)),
        ],
        out_shape=[
            jax.ShapeDtypeStruct((T, 1), jnp.int32),
            jax.ShapeDtypeStruct((T, 1), jnp.int32),
        ],
    )(e0, e1, r0, r1, offs)

    p0f = p0.reshape(T)
    p1f = p1.reshape(T)
    bmapf = bmap.reshape(NBLKP)

    mesh = plsc.VectorSubcoreMesh(core_axis_name="c", subcore_axis_name="s")

    xs = pl.kernel(
        _sc_dispatch_body,
        out_type=jax.ShapeDtypeStruct((NROWS, D), jnp.float32),
        mesh=mesh,
        scratch_types=[
            pltpu.VMEM((CH, D), jnp.float32),
            pltpu.VMEM((CH,), jnp.int32),
            pltpu.VMEM((CH,), jnp.int32),
            pltpu.SemaphoreType.DMA,
        ],
    )(x, p0f, p1f)

    ys = pl.pallas_call(
        _moe_body,
        grid_spec=pltpu.PrefetchScalarGridSpec(
            num_scalar_prefetch=1,
            grid=(NBLK,),
            in_specs=[
                pl.BlockSpec((BTM, D), lambda b, m: (b, 0)),
                pl.BlockSpec((1, FF, D), lambda b, m: (m[b], 0, 0)),
                pl.BlockSpec((1, FF, D), lambda b, m: (m[b], 0, 0)),
                pl.BlockSpec((1, D, FF), lambda b, m: (m[b], 0, 0)),
            ],
            out_specs=pl.BlockSpec((BTM, D), lambda b, m: (b, 0)),
        ),
        out_shape=jax.ShapeDtypeStruct((NROWS, D), jnp.float32),
        compiler_params=pltpu.CompilerParams(
            dimension_semantics=("arbitrary",)),
    )(bmapf, xs, w1, w3, w2)

    y0, y1 = pl.kernel(
        _sc_combine_body,
        out_type=[
            jax.ShapeDtypeStruct((T, D), jnp.float32),
            jax.ShapeDtypeStruct((T, D), jnp.float32),
        ],
        mesh=mesh,
        scratch_types=[
            pltpu.VMEM((CH, D), jnp.float32),
            pltpu.VMEM((CH,), jnp.int32),
            pltpu.VMEM((CH,), jnp.int32),
            pltpu.SemaphoreType.DMA,
        ],
    )(ys, p0f, p1f)

    BTC = 512
    out = pl.pallas_call(
        _combine_body,
        grid=(T // BTC,),
        in_specs=[
            pl.BlockSpec((BTC, D), lambda t: (t, 0)),
            pl.BlockSpec((BTC, D), lambda t: (t, 0)),
            pl.BlockSpec((BTC, 1), lambda t: (t, 0)),
            pl.BlockSpec((BTC, 1), lambda t: (t, 0)),
        ],
        out_specs=pl.BlockSpec((BTC, D), lambda t: (t, 0)),
        out_shape=jax.ShapeDtypeStruct((T, D), jnp.float32),
    )(y0, y1, w0, w1c)
    return out


# speedup vs baseline: 1.0623x; 1.0623x over previous
import functools

import jax
import jax.numpy as jnp
from jax import lax
from jax.experimental import pallas as pl
from jax.experimental.pallas import tpu as pltpu
from jax.experimental.pallas import tpu_sc as plsc

BTM = 256
CH = 16


def _routing_body(x_ref, gw_ref, w0_ref, w1_ref, p0_ref, p1_ref, bmap_ref,
                  cnt_ref, e0s_ref, e1s_ref, r0s_ref, r1s_ref):
    s = pl.program_id(0)
    NS = pl.num_programs(0)
    S = x_ref.shape[0]
    E = gw_ref.shape[0]
    NBLKP = bmap_ref.shape[1]
    T = e0s_ref.shape[0]

    @pl.when(s == 0)
    def _():
        cnt_ref[...] = jnp.zeros_like(cnt_ref)

    x = x_ref[...]
    logits = lax.dot_general(x, gw_ref[...], (((1,), (1,)), ((), ())),
                             preferred_element_type=jnp.float32)
    iota = lax.broadcasted_iota(jnp.int32, (S, E), 1)
    m = jnp.max(logits, axis=-1, keepdims=True)
    p = jnp.exp(logits - m)
    m1 = jnp.max(p, axis=-1, keepdims=True)
    i1 = jnp.min(jnp.where(p == m1, iota, E), axis=-1, keepdims=True)
    mask1 = iota == i1
    p2 = jnp.where(mask1, -jnp.inf, p)
    m2 = jnp.max(p2, axis=-1, keepdims=True)
    i2 = jnp.min(jnp.where(p2 == m2, iota, E), axis=-1, keepdims=True)
    mask2 = iota == i2
    denom = m1 + m2
    w0_ref[...] = m1 / denom
    w1_ref[...] = m2 / denom

    oh = mask1.astype(jnp.float32) + mask2.astype(jnp.float32)
    tril = (lax.broadcasted_iota(jnp.int32, (S, S), 0) >
            lax.broadcasted_iota(jnp.int32, (S, S), 1)).astype(jnp.float32)
    csum = lax.dot_general(tril, oh, (((1,), (0,)), ((), ())),
                           preferred_element_type=jnp.float32)
    csum = csum + cnt_ref[...].astype(jnp.float32)
    r0 = jnp.sum(jnp.where(mask1, csum, 0.0), axis=-1,
                 keepdims=True).astype(jnp.int32)
    r1 = jnp.sum(jnp.where(mask2, csum, 0.0), axis=-1,
                 keepdims=True).astype(jnp.int32)

    sl = pl.ds(s * S, S)
    e0s_ref[sl] = i1
    e1s_ref[sl] = i2
    r0s_ref[sl] = r0
    r1s_ref[sl] = r1
    cnt_ref[...] = (cnt_ref[...] +
                    jnp.sum(oh, axis=0, keepdims=True).astype(jnp.int32))

    @pl.when(s == NS - 1)
    def _():
        cnt = cnt_ref[...]
        padded = ((cnt + (BTM - 1)) // BTM) * BTM
        upper = (lax.broadcasted_iota(jnp.int32, (E, E), 0) <
                 lax.broadcasted_iota(jnp.int32, (E, E), 1)).astype(jnp.float32)
        offs = lax.dot_general(padded.astype(jnp.float32), upper,
                               (((1,), (0,)), ((), ())),
                               preferred_element_type=jnp.float32)
        iota_te = lax.broadcasted_iota(jnp.int32, (T, E), 1)
        mask0 = (iota_te == e0s_ref[...]).astype(jnp.float32)
        mask1a = (iota_te == e1s_ref[...]).astype(jnp.float32)
        base0 = lax.dot_general(mask0, offs, (((1,), (1,)), ((), ())),
                                preferred_element_type=jnp.float32)
        base1 = lax.dot_general(mask1a, offs, (((1,), (1,)), ((), ())),
                                preferred_element_type=jnp.float32)
        p0_ref[...] = base0.astype(jnp.int32) + r0s_ref[...]
        p1_ref[...] = base1.astype(jnp.int32) + r1s_ref[...]

        incl = offs + padded.astype(jnp.float32)
        total = jnp.sum(padded)
        lane = lax.broadcasted_iota(jnp.int32, (1, NBLKP), 1)
        pos = jnp.minimum(lane * BTM, total - BTM).astype(jnp.float32)
        bexp = jnp.zeros((1, NBLKP), jnp.int32)
        for e in range(E):
            bexp = bexp + (pos >= incl[:, e:e + 1]).astype(jnp.int32)
        nblk_used = (total // BTM).astype(jnp.int32)
        bmap_ref[...] = jnp.where(lane == NBLKP - 1, nblk_used, bexp)


def _moe_body(bmap_ref, xs_ref, w1_ref, w3_ref, w2_ref, o_ref):
    b = pl.program_id(0)
    nused = bmap_ref[bmap_ref.shape[0] - 1]

    @pl.when(b < nused)
    def _():
        x = xs_ref[...]
        g = lax.dot_general(x, w1_ref[0], (((1,), (1,)), ((), ())),
                            preferred_element_type=jnp.float32)
        u = lax.dot_general(x, w3_ref[0], (((1,), (1,)), ((), ())),
                            preferred_element_type=jnp.float32)
        h = g * lax.logistic(g) * u
        o_ref[...] = lax.dot_general(h, w2_ref[0], (((1,), (1,)), ((), ())),
                                     preferred_element_type=jnp.float32)


def _combine_body(y0_ref, y1_ref, w0_ref, w1_ref, o_ref):
    o_ref[...] = w0_ref[...] * y0_ref[...] + w1_ref[...] * y1_ref[...]


def _sc_worker_id():
    return lax.axis_index("s") * 2 + lax.axis_index("c")


def _sc_dispatch_body(x_hbm, p0_hbm, p1_hbm, xs_hbm, rows_a, rows_b,
                      p0_v, p1_v, lsem, ssem):
    T = x_hbm.shape[0]
    wid = _sc_worker_id()
    per_w = T // 32
    nch = per_w // CH
    base = wid * per_w
    pltpu.sync_copy(p0_hbm.at[pl.ds(base, per_w)], p0_v)
    pltpu.sync_copy(p1_hbm.at[pl.ds(base, per_w)], p1_v)
    bufs = [rows_a, rows_b]
    loads = {0: pltpu.async_copy(x_hbm.at[pl.ds(base, CH)], rows_a, lsem)}
    scats = {}
    for ci in range(nch):
        if ci >= 1:
            scats[ci - 1][0].wait()
            scats[ci - 1][1].wait()
        if ci + 1 < nch:
            loads[ci + 1] = pltpu.async_copy(
                x_hbm.at[pl.ds(base + (ci + 1) * CH, CH)],
                bufs[(ci + 1) % 2], lsem)
        loads[ci].wait()
        sl = pl.ds(ci * CH, CH)
        i0 = p0_v[sl]
        i1 = p1_v[sl]
        buf = bufs[ci % 2]
        scats[ci] = (pltpu.async_copy(buf, xs_hbm.at[i0], ssem),
                     pltpu.async_copy(buf, xs_hbm.at[i1], ssem))
    scats[nch - 1][0].wait()
    scats[nch - 1][1].wait()


def _sc_combine_body(ys_hbm, p0_hbm, p1_hbm, y0_hbm, y1_hbm, rows_a, rows_b,
                     p0_v, p1_v, g0sem, g1sem, s0sem, s1sem):
    T = y0_hbm.shape[0]
    wid = _sc_worker_id()
    per_w = T // 32
    nch = per_w // CH
    base = wid * per_w
    pltpu.sync_copy(p0_hbm.at[pl.ds(base, per_w)], p0_v)
    pltpu.sync_copy(p1_hbm.at[pl.ds(base, per_w)], p1_v)

    g0 = pltpu.async_copy(ys_hbm.at[p0_v[pl.ds(0, CH)]], rows_a, g0sem)
    s1 = None
    for ci in range(nch):
        dst = pl.ds(base + ci * CH, CH)
        g0.wait()
        s0 = pltpu.async_copy(rows_a, y0_hbm.at[dst], s0sem)
        if s1 is not None:
            s1.wait()
        g1 = pltpu.async_copy(ys_hbm.at[p1_v[pl.ds(ci * CH, CH)]],
                              rows_b, g1sem)
        s0.wait()
        if ci + 1 < nch:
            g0 = pltpu.async_copy(ys_hbm.at[p0_v[pl.ds((ci + 1) * CH, CH)]],
                                  rows_a, g0sem)
        g1.wait()
        s1 = pltpu.async_copy(rows_b, y1_hbm.at[dst], s1sem)
    s1.wait()


def kernel(hidden_states, gate_w, w1, w2, w3, num_global_tokens,
           max_num_tokens_per_gpu):
    T, D = hidden_states.shape
    E, FF, _ = w1.shape
    K = 2
    x = hidden_states.astype(jnp.float32)

    NBLK = (T * K) // BTM + E
    NBLKP = NBLK + 1
    NROWS = NBLK * BTM
    S = T // 2

    w0, w1c, p0, p1, bmap = pl.pallas_call(
        _routing_body,
        grid=(T // S,),
        in_specs=[
            pl.BlockSpec((S, D), lambda s: (s, 0)),
            pl.BlockSpec((E, D), lambda s: (0, 0)),
        ],
        out_specs=[
            pl.BlockSpec((S, 1), lambda s: (s, 0)),
            pl.BlockSpec((S, 1), lambda s: (s, 0)),
            pl.BlockSpec((T, 1), lambda s: (0, 0)),
            pl.BlockSpec((T, 1), lambda s: (0, 0)),
            pl.BlockSpec((1, NBLKP), lambda s: (0, 0)),
        ],
        out_shape=[
            jax.ShapeDtypeStruct((T, 1), jnp.float32),
            jax.ShapeDtypeStruct((T, 1), jnp.float32),
            jax.ShapeDtypeStruct((T, 1), jnp.int32),
            jax.ShapeDtypeStruct((T, 1), jnp.int32),
            jax.ShapeDtypeStruct((1, NBLKP), jnp.int32),
        ],
        scratch_shapes=[
            pltpu.VMEM((1, E), jnp.int32),
            pltpu.VMEM((T, 1), jnp.int32),
            pltpu.VMEM((T, 1), jnp.int32),
            pltpu.VMEM((T, 1), jnp.int32),
            pltpu.VMEM((T, 1), jnp.int32),
        ],
        compiler_params=pltpu.CompilerParams(
            dimension_semantics=("arbitrary",)),
    )(x, gate_w)

    p0f = p0.reshape(T)
    p1f = p1.reshape(T)
    PW = T // 32

    mesh = plsc.VectorSubcoreMesh(core_axis_name="c", subcore_axis_name="s")

    xs = pl.kernel(
        _sc_dispatch_body,
        out_type=jax.ShapeDtypeStruct((NROWS, D), jnp.float32),
        mesh=mesh,
        scratch_types=[
            pltpu.VMEM((CH, D), jnp.float32),
            pltpu.VMEM((CH, D), jnp.float32),
            pltpu.VMEM((PW,), jnp.int32),
            pltpu.VMEM((PW,), jnp.int32),
            pltpu.SemaphoreType.DMA,
            pltpu.SemaphoreType.DMA,
        ],
    )(x, p0f, p1f)

    ys = pl.pallas_call(
        _moe_body,
        grid_spec=pltpu.PrefetchScalarGridSpec(
            num_scalar_prefetch=1,
            grid=(NBLK,),
            in_specs=[
                pl.BlockSpec((BTM, D), lambda b, m: (b, 0)),
                pl.BlockSpec((1, FF, D), lambda b, m: (m[b], 0, 0)),
                pl.BlockSpec((1, FF, D), lambda b, m: (m[b], 0, 0)),
                pl.BlockSpec((1, D, FF), lambda b, m: (m[b], 0, 0)),
            ],
            out_specs=pl.BlockSpec((BTM, D), lambda b, m: (b, 0)),
        ),
        out_shape=jax.ShapeDtypeStruct((NROWS, D), jnp.float32),
        compiler_params=pltpu.CompilerParams(
            dimension_semantics=("arbitrary",)),
    )(bmap.reshape(NBLKP), xs, w1, w3, w2)

    y0, y1 = pl.kernel(
        _sc_combine_body,
        out_type=[
            jax.ShapeDtypeStruct((T, D), jnp.float32),
            jax.ShapeDtypeStruct((T, D), jnp.float32),
        ],
        mesh=mesh,
        scratch_types=[
            pltpu.VMEM((CH, D), jnp.float32),
            pltpu.VMEM((CH, D), jnp.float32),
            pltpu.VMEM((PW,), jnp.int32),
            pltpu.VMEM((PW,), jnp.int32),
            pltpu.SemaphoreType.DMA,
            pltpu.SemaphoreType.DMA,
            pltpu.SemaphoreType.DMA,
            pltpu.SemaphoreType.DMA,
        ],
    )(ys, p0f, p1f)

    BTC = 512
    out = pl.pallas_call(
        _combine_body,
        grid=(T // BTC,),
        in_specs=[
            pl.BlockSpec((BTC, D), lambda t: (t, 0)),
            pl.BlockSpec((BTC, D), lambda t: (t, 0)),
            pl.BlockSpec((BTC, 1), lambda t: (t, 0)),
            pl.BlockSpec((BTC, 1), lambda t: (t, 0)),
        ],
        out_specs=pl.BlockSpec((BTC, D), lambda t: (t, 0)),
        out_shape=jax.ShapeDtypeStruct((T, D), jnp.float32),
    )(y0, y1, w0, w1c)
    return out
